# baseline (speedup 1.0000x reference)
import functools

import jax
import jax.numpy as jnp
from jax import lax
from jax.experimental import pallas as pl
from jax.experimental.pallas import tpu as pltpu
from jax.experimental.pallas import tpu_sc as plsc

_EPS = 1e-09


@functools.lru_cache(maxsize=None)
def _make_sc_gather(NT, F, R):
    info = plsc.get_sparse_core_info()
    NW = info.num_cores * info.num_subcores
    NC = info.num_cores
    rows_per_w = R // NW
    assert rows_per_w * NW == R
    C = 3200
    assert rows_per_w % C == 0
    n_chunks = rows_per_w // C

    mesh = plsc.VectorSubcoreMesh(core_axis_name="c", subcore_axis_name="s")

    @functools.partial(
        pl.kernel,
        mesh=mesh,
        compiler_params=pltpu.CompilerParams(use_tc_tiling_on_sc=False),
        out_type=[
            jax.ShapeDtypeStruct((R, F), jnp.float32),
            jax.ShapeDtypeStruct((R, F), jnp.float32),
        ],
        scratch_types=[
            pltpu.VMEM((C,), jnp.int32),
            pltpu.VMEM((C, F), jnp.float32),
            pltpu.VMEM((C, F), jnp.float32),
            pltpu.SemaphoreType.DMA,
        ],
    )
    def sc_gather(means_hbm, vars_hbm, idx_hbm, means_out, vars_out,
                  idx_v, m_v, v_v, sem):
        wid = lax.axis_index("s") * NC + lax.axis_index("c")
        base = wid * rows_per_w

        def body(ci, carry):
            off = base + ci * C
            pltpu.sync_copy(idx_hbm.at[pl.ds(off, C)], idx_v)
            cm = pltpu.async_copy(means_hbm.at[idx_v], m_v, sem)
            cv = pltpu.async_copy(vars_hbm.at[idx_v], v_v, sem)
            cm.wait()
            cv.wait()
            pltpu.sync_copy(m_v, means_out.at[pl.ds(off, C)])
            pltpu.sync_copy(v_v, vars_out.at[pl.ds(off, C)])
            return carry

        lax.fori_loop(0, n_chunks, body, 0)

    return sc_gather


def _normalize_body(ts_ref, m_ref, v_ref, o_ref):
    o_ref[...] = (ts_ref[...] - m_ref[...]) / (jnp.sqrt(v_ref[...]) + _EPS)


@functools.lru_cache(maxsize=None)
def _make_tc_normalize(rows, cols, block_rows):
    grid = (rows // block_rows,)
    spec = pl.BlockSpec((block_rows, cols), lambda i: (i, 0))
    return pl.pallas_call(
        _normalize_body,
        grid=grid,
        in_specs=[spec, spec, spec],
        out_specs=spec,
        out_shape=jax.ShapeDtypeStruct((rows, cols), jnp.float32),
    )


def kernel(ts_index, window_indices, ts, means_table, vars_table):
    B, L = window_indices.shape
    N, T, F = means_table.shape
    R = B * L

    flat_idx = (ts_index.astype(jnp.int32)[:, None] * T
                + window_indices.astype(jnp.int32)).reshape(R)
    means_flat = means_table.reshape(N * T, F)
    vars_flat = vars_table.reshape(N * T, F)

    means_rows, vars_rows = _make_sc_gather(N * T, F, R)(
        means_flat, vars_flat, flat_idx)

    cols = 1024
    rows = (R * F) // cols
    norm = _make_tc_normalize(rows, cols, 512)(
        ts.reshape(rows, cols),
        means_rows.reshape(rows, cols),
        vars_rows.reshape(rows, cols),
    )

    return (norm.reshape(B, L, F),
            means_rows.reshape(B, L, F),
            vars_rows.reshape(B, L, F))


# speedup vs baseline: 1.0131x; 1.0131x over previous
import functools

import jax
import jax.numpy as jnp
from jax import lax
from jax.experimental import pallas as pl
from jax.experimental.pallas import tpu as pltpu
from jax.experimental.pallas import tpu_sc as plsc

_EPS = 1e-09


def _table_transpose_body(m_ref, v_ref, mo_ref, vo_ref):
    sb = m_ref.shape[0]
    rows = sb * m_ref.shape[2]
    mo_ref[...] = jnp.transpose(m_ref[...], (0, 2, 1)).reshape(rows, 16)
    vo_ref[...] = jnp.transpose(v_ref[...], (0, 2, 1)).reshape(rows, 16)


@functools.lru_cache(maxsize=None)
def _make_table_transpose(N, T, F, SB):
    grid = (N // SB,)
    in_spec = pl.BlockSpec((SB, F, T), lambda i: (i, 0, 0))
    out_spec = pl.BlockSpec((SB * T, F), lambda i: (i, 0))
    return pl.pallas_call(
        _table_transpose_body,
        grid=grid,
        in_specs=[in_spec, in_spec],
        out_specs=[out_spec, out_spec],
        out_shape=[jax.ShapeDtypeStruct((N * T, F), jnp.float32),
                   jax.ShapeDtypeStruct((N * T, F), jnp.float32)],
    )


@functools.lru_cache(maxsize=None)
def _make_sc_gather(NT, F, R):
    info = plsc.get_sparse_core_info()
    NW = info.num_cores * info.num_subcores
    NC = info.num_cores
    rows_per_w = R // NW
    assert rows_per_w * NW == R
    C = 3200
    assert rows_per_w % C == 0
    n_chunks = rows_per_w // C

    mesh = plsc.VectorSubcoreMesh(core_axis_name="c", subcore_axis_name="s")

    @functools.partial(
        pl.kernel,
        mesh=mesh,
        compiler_params=pltpu.CompilerParams(use_tc_tiling_on_sc=False),
        out_type=[
            jax.ShapeDtypeStruct((R, F), jnp.float32),
            jax.ShapeDtypeStruct((R, F), jnp.float32),
        ],
        scratch_types=[
            pltpu.VMEM((C,), jnp.int32),
            pltpu.VMEM((C, F), jnp.float32),
            pltpu.VMEM((C, F), jnp.float32),
            pltpu.SemaphoreType.DMA,
        ],
    )
    def sc_gather(means_hbm, vars_hbm, idx_hbm, means_out, vars_out,
                  idx_v, m_v, v_v, sem):
        wid = lax.axis_index("s") * NC + lax.axis_index("c")
        base = wid * rows_per_w

        def body(ci, carry):
            off = base + ci * C
            pltpu.sync_copy(idx_hbm.at[pl.ds(off, C)], idx_v)
            cm = pltpu.async_copy(means_hbm.at[idx_v], m_v, sem)
            cv = pltpu.async_copy(vars_hbm.at[idx_v], v_v, sem)
            cm.wait()
            cv.wait()
            pltpu.sync_copy(m_v, means_out.at[pl.ds(off, C)])
            pltpu.sync_copy(v_v, vars_out.at[pl.ds(off, C)])
            return carry

        lax.fori_loop(0, n_chunks, body, 0)

    return sc_gather


def _normalize_body(ts_ref, m_ref, v_ref, o_ref, mo_ref, vo_ref):
    m_t = jnp.transpose(m_ref[...], (1, 0))[None]
    v_t = jnp.transpose(v_ref[...], (1, 0))[None]
    mo_ref[...] = m_t
    vo_ref[...] = v_t
    o_ref[...] = (ts_ref[...] - m_t) / (jnp.sqrt(v_t) + _EPS)


@functools.lru_cache(maxsize=None)
def _make_normalize(L, F, B):
    grid = (L,)
    ts_spec = pl.BlockSpec((1, F, B), lambda i: (i, 0, 0))
    g_spec = pl.BlockSpec((B, F), lambda i: (i, 0))
    shape = jax.ShapeDtypeStruct((L, F, B), jnp.float32)
    return pl.pallas_call(
        _normalize_body,
        grid=grid,
        in_specs=[ts_spec, g_spec, g_spec],
        out_specs=[ts_spec, ts_spec, ts_spec],
        out_shape=[shape, shape, shape],
    )


def kernel(ts_index, window_indices, ts, means_table, vars_table):
    B, L = window_indices.shape
    N, T, F = means_table.shape
    R = B * L

    mt_nat = jnp.transpose(means_table, (0, 2, 1))
    vt_nat = jnp.transpose(vars_table, (0, 2, 1))
    means_flat, vars_flat = _make_table_transpose(N, T, F, 8)(mt_nat, vt_nat)

    flat_idx = (ts_index.astype(jnp.int32)[None, :] * T
                + jnp.transpose(window_indices).astype(jnp.int32)).reshape(R)

    means_rows, vars_rows = _make_sc_gather(N * T, F, R)(
        means_flat, vars_flat, flat_idx)

    ts_p = jnp.transpose(ts, (1, 2, 0))
    norm_p, means_p, vars_p = _make_normalize(L, F, B)(
        ts_p, means_rows, vars_rows)

    return (jnp.transpose(norm_p, (2, 0, 1)),
            jnp.transpose(means_p, (2, 0, 1)),
            jnp.transpose(vars_p, (2, 0, 1)))


# speedup vs baseline: 3.5592x; 3.5130x over previous
import functools

import jax
import jax.numpy as jnp
from jax import lax
from jax.experimental import pallas as pl
from jax.experimental.pallas import tpu as pltpu
from jax.experimental.pallas import tpu_sc as plsc

_EPS = 1e-09


def _table_repack_body(m_ref, v_ref, eye_ref, mo_ref, vo_ref):
    SB, F, T = m_ref.shape
    dn = (((0,), (0,)), ((), ()))

    def repack(x):
        x2 = x.reshape(SB * F, T)
        return lax.dot_general(x2, eye_ref[...], dn,
                               precision=lax.Precision.HIGHEST,
                               preferred_element_type=jnp.float32)

    mo_ref[...] = repack(m_ref[...])
    vo_ref[...] = repack(v_ref[...])


@functools.lru_cache(maxsize=None)
def _make_table_repack(N, T, F, SB):
    grid = (N // SB,)
    in_spec = pl.BlockSpec((SB, F, T), lambda i: (i, 0, 0))
    eye_spec = pl.BlockSpec((SB * F, SB * F), lambda i: (0, 0))
    out_spec = pl.BlockSpec((T, SB * F), lambda i: (i, 0))
    out_shape = jax.ShapeDtypeStruct((N * T * F // (SB * F), SB * F),
                                     jnp.float32)
    return pl.pallas_call(
        _table_repack_body,
        grid=grid,
        in_specs=[in_spec, in_spec, eye_spec],
        out_specs=[out_spec, out_spec],
        out_shape=[out_shape, out_shape],
    )


@functools.lru_cache(maxsize=None)
def _make_sc_gather(NT, F, R):
    info = plsc.get_sparse_core_info()
    NW = info.num_cores * info.num_subcores
    NC = info.num_cores
    rows_per_w = R // NW
    assert rows_per_w * NW == R
    C = 3200
    assert rows_per_w % C == 0
    n_chunks = rows_per_w // C

    mesh = plsc.VectorSubcoreMesh(core_axis_name="c", subcore_axis_name="s")

    @functools.partial(
        pl.kernel,
        mesh=mesh,
        compiler_params=pltpu.CompilerParams(use_tc_tiling_on_sc=False),
        out_type=[
            jax.ShapeDtypeStruct((R, F), jnp.float32),
            jax.ShapeDtypeStruct((R, F), jnp.float32),
        ],
        scratch_types=[
            pltpu.VMEM((C,), jnp.int32),
            pltpu.VMEM((C, F), jnp.float32),
            pltpu.VMEM((C, F), jnp.float32),
            pltpu.SemaphoreType.DMA,
        ],
    )
    def sc_gather(means_hbm, vars_hbm, idx_hbm, means_out, vars_out,
                  idx_v, m_v, v_v, sem):
        wid = lax.axis_index("s") * NC + lax.axis_index("c")
        base = wid * rows_per_w

        def body(ci, carry):
            off = base + ci * C
            pltpu.sync_copy(idx_hbm.at[pl.ds(off, C)], idx_v)
            cm = pltpu.async_copy(means_hbm.at[idx_v], m_v, sem)
            cv = pltpu.async_copy(vars_hbm.at[idx_v], v_v, sem)
            cm.wait()
            cv.wait()
            pltpu.sync_copy(m_v, means_out.at[pl.ds(off, C)])
            pltpu.sync_copy(v_v, vars_out.at[pl.ds(off, C)])
            return carry

        lax.fori_loop(0, n_chunks, body, 0)

    return sc_gather


def _normalize_body(ts_ref, m_ref, v_ref, eye_ref, o_ref, mo_ref, vo_ref):
    _, F, B = ts_ref.shape
    G = B // 64
    dn = (((0,), (0,)), ((), ()))

    def tr(x):
        z = lax.dot_general(x, eye_ref[...], dn,
                            precision=lax.Precision.HIGHEST,
                            preferred_element_type=jnp.float32)
        z3 = z.reshape(64, F, G)
        return jnp.transpose(z3, (1, 0, 2)).reshape(1, F, B)

    m_t = tr(m_ref[...].reshape(G, 1024))
    v_t = tr(v_ref[...].reshape(G, 1024))
    mo_ref[...] = m_t
    vo_ref[...] = v_t
    o_ref[...] = (ts_ref[...] - m_t) / (jnp.sqrt(v_t) + _EPS)


@functools.lru_cache(maxsize=None)
def _make_normalize(L, F, B):
    grid = (L,)
    G = B // 64
    ts_spec = pl.BlockSpec((1, F, B), lambda i: (i, 0, 0))
    g_spec = pl.BlockSpec((1, G, 1024), lambda i: (i, 0, 0))
    eye_spec = pl.BlockSpec((G, G), lambda i: (0, 0))
    shape = jax.ShapeDtypeStruct((L, F, B), jnp.float32)
    return pl.pallas_call(
        _normalize_body,
        grid=grid,
        in_specs=[ts_spec, g_spec, g_spec, eye_spec],
        out_specs=[ts_spec, ts_spec, ts_spec],
        out_shape=[shape, shape, shape],
    )


def kernel(ts_index, window_indices, ts, means_table, vars_table):
    B, L = window_indices.shape
    N, T, F = means_table.shape
    R = B * L
    SB = 128 // F

    mt_nat = jnp.transpose(means_table, (0, 2, 1))
    vt_nat = jnp.transpose(vars_table, (0, 2, 1))
    eye128 = jnp.eye(SB * F, dtype=jnp.float32)
    means_w, vars_w = _make_table_repack(N, T, F, SB)(mt_nat, vt_nat, eye128)
    means_flat = means_w.reshape(N * T, F)
    vars_flat = vars_w.reshape(N * T, F)

    s = ts_index.astype(jnp.int32)
    t = jnp.transpose(window_indices).astype(jnp.int32)
    g = ((s // SB) * (T * SB) + (s % SB))[None, :] + t * SB
    g_pi = jnp.transpose(g.reshape(L, 64, B // 64), (0, 2, 1)).reshape(L, B)
    flat_idx = g_pi.reshape(R)

    means_rows, vars_rows = _make_sc_gather(N * T, F, R)(
        means_flat, vars_flat, flat_idx)

    ts_p = jnp.transpose(ts, (1, 2, 0))
    eye64 = jnp.eye(B // 64, dtype=jnp.float32)
    norm_p, means_p, vars_p = _make_normalize(L, F, B)(
        ts_p,
        means_rows.reshape(L, B // 64, 1024),
        vars_rows.reshape(L, B // 64, 1024),
        eye64)

    return (jnp.transpose(norm_p, (2, 0, 1)),
            jnp.transpose(means_p, (2, 0, 1)),
            jnp.transpose(vars_p, (2, 0, 1)))


# speedup vs baseline: 3.9103x; 1.0986x over previous
import functools

import jax
import jax.numpy as jnp
from jax import lax
from jax.experimental import pallas as pl
from jax.experimental.pallas import tpu as pltpu
from jax.experimental.pallas import tpu_sc as plsc

_EPS = 1e-09


def _table_repack_body(m_ref, v_ref, eye_ref, mo_ref, vo_ref):
    SB, F, T = m_ref.shape
    dn = (((0,), (0,)), ((), ()))

    def repack(x):
        x2 = x.reshape(SB * F, T)
        return lax.dot_general(x2, eye_ref[...], dn,
                               precision=lax.Precision.HIGHEST,
                               preferred_element_type=jnp.float32)

    mo_ref[...] = repack(m_ref[...])
    vo_ref[...] = repack(v_ref[...])


@functools.lru_cache(maxsize=None)
def _make_table_repack(N, T, F, SB):
    grid = (N // SB,)
    in_spec = pl.BlockSpec((SB, F, T), lambda i: (i, 0, 0))
    eye_spec = pl.BlockSpec((SB * F, SB * F), lambda i: (0, 0))
    out_spec = pl.BlockSpec((T, SB * F), lambda i: (i, 0))
    out_shape = jax.ShapeDtypeStruct((N * T * F // (SB * F), SB * F),
                                     jnp.float32)
    return pl.pallas_call(
        _table_repack_body,
        grid=grid,
        in_specs=[in_spec, in_spec, eye_spec],
        out_specs=[out_spec, out_spec],
        out_shape=[out_shape, out_shape],
    )


@functools.lru_cache(maxsize=None)
def _make_sc_gather(NT, F, R):
    info = plsc.get_sparse_core_info()
    NW = info.num_cores * info.num_subcores
    NC = info.num_cores
    rows_per_w = R // NW
    assert rows_per_w * NW == R
    C = 3200
    assert rows_per_w % C == 0
    n_chunks = rows_per_w // C

    mesh = plsc.VectorSubcoreMesh(core_axis_name="c", subcore_axis_name="s")

    @functools.partial(
        pl.kernel,
        mesh=mesh,
        compiler_params=pltpu.CompilerParams(use_tc_tiling_on_sc=False),
        out_type=[
            jax.ShapeDtypeStruct((R, F), jnp.float32),
            jax.ShapeDtypeStruct((R, F), jnp.float32),
        ],
        scratch_types=[
            pltpu.VMEM((C,), jnp.int32),
            pltpu.VMEM((C, F), jnp.float32),
            pltpu.VMEM((C, F), jnp.float32),
            pltpu.SemaphoreType.DMA,
        ],
    )
    def sc_gather(means_hbm, vars_hbm, idx_hbm, means_out, vars_out,
                  idx_v, m_v, v_v, sem):
        wid = lax.axis_index("s") * NC + lax.axis_index("c")
        base = wid * rows_per_w

        def body(ci, carry):
            off = base + ci * C
            pltpu.sync_copy(idx_hbm.at[pl.ds(off, C)], idx_v)
            cm = pltpu.async_copy(means_hbm.at[idx_v], m_v, sem)
            cv = pltpu.async_copy(vars_hbm.at[idx_v], v_v, sem)
            cm.wait()
            cv.wait()
            pltpu.sync_copy(m_v, means_out.at[pl.ds(off, C)])
            pltpu.sync_copy(v_v, vars_out.at[pl.ds(off, C)])
            return carry

        lax.fori_loop(0, n_chunks, body, 0)

    return sc_gather


def _normalize_body(ts_ref, m_ref, v_ref, eye_ref, o_ref, mo_ref, vo_ref):
    LB, F, B = ts_ref.shape
    G = B // 64
    dn = (((0,), (0,)), ((), ()))

    def tr(x):
        z = lax.dot_general(x, eye_ref[...], dn,
                            precision=lax.Precision.HIGHEST,
                            preferred_element_type=jnp.float32)
        z5 = z.reshape(64, F, LB, G)
        return jnp.transpose(z5, (2, 1, 0, 3)).reshape(LB, F, B)

    m_t = tr(m_ref[...].reshape(LB * G, 1024))
    v_t = tr(v_ref[...].reshape(LB * G, 1024))
    mo_ref[...] = m_t
    vo_ref[...] = v_t
    o_ref[...] = (ts_ref[...] - m_t) / (jnp.sqrt(v_t) + _EPS)


@functools.lru_cache(maxsize=None)
def _make_normalize(L, F, B, LB):
    grid = (L // LB,)
    G = B // 64
    ts_spec = pl.BlockSpec((LB, F, B), lambda i: (i, 0, 0))
    g_spec = pl.BlockSpec((LB, G, 1024), lambda i: (i, 0, 0))
    eye_spec = pl.BlockSpec((LB * G, LB * G), lambda i: (0, 0))
    shape = jax.ShapeDtypeStruct((L, F, B), jnp.float32)
    return pl.pallas_call(
        _normalize_body,
        grid=grid,
        in_specs=[ts_spec, g_spec, g_spec, eye_spec],
        out_specs=[ts_spec, ts_spec, ts_spec],
        out_shape=[shape, shape, shape],
    )


def kernel(ts_index, window_indices, ts, means_table, vars_table):
    B, L = window_indices.shape
    N, T, F = means_table.shape
    R = B * L
    SB = 128 // F

    mt_nat = jnp.transpose(means_table, (0, 2, 1))
    vt_nat = jnp.transpose(vars_table, (0, 2, 1))
    eye128 = jnp.eye(SB * F, dtype=jnp.float32)
    means_w, vars_w = _make_table_repack(N, T, F, SB)(mt_nat, vt_nat, eye128)
    means_flat = means_w.reshape(N * T, F)
    vars_flat = vars_w.reshape(N * T, F)

    s = ts_index.astype(jnp.int32)
    t = jnp.transpose(window_indices).astype(jnp.int32)
    g = ((s // SB) * (T * SB) + (s % SB))[None, :] + t * SB
    g_pi = jnp.transpose(g.reshape(L, 64, B // 64), (0, 2, 1)).reshape(L, B)
    flat_idx = g_pi.reshape(R)

    means_rows, vars_rows = _make_sc_gather(N * T, F, R)(
        means_flat, vars_flat, flat_idx)

    ts_p = jnp.transpose(ts, (1, 2, 0))
    LB = 4
    eyeN = jnp.eye(LB * B // 64, dtype=jnp.float32)
    norm_p, means_p, vars_p = _make_normalize(L, F, B, LB)(
        ts_p,
        means_rows.reshape(L, B // 64, 1024),
        vars_rows.reshape(L, B // 64, 1024),
        eyeN)

    return (jnp.transpose(norm_p, (2, 0, 1)),
            jnp.transpose(means_p, (2, 0, 1)),
            jnp.transpose(vars_p, (2, 0, 1)))


# speedup vs baseline: 4.8832x; 1.2488x over previous
import functools

import jax
import jax.numpy as jnp
from jax import lax
from jax.experimental import pallas as pl
from jax.experimental.pallas import tpu as pltpu
from jax.experimental.pallas import tpu_sc as plsc

_EPS = 1e-09


def _table_repack_body(m_ref, v_ref, mo_ref, vo_ref):
    SB, F, T = m_ref.shape
    def repack(x):
        x2 = x.reshape(SB * F, T)
        return jnp.transpose(x2)

    mo_ref[...] = repack(m_ref[...])
    vo_ref[...] = repack(v_ref[...])


@functools.lru_cache(maxsize=None)
def _make_table_repack(N, T, F, SB):
    grid = (N // SB,)
    in_spec = pl.BlockSpec((SB, F, T), lambda i: (i, 0, 0))
    out_spec = pl.BlockSpec((T, SB * F), lambda i: (i, 0))
    out_shape = jax.ShapeDtypeStruct((N * T * F // (SB * F), SB * F),
                                     jnp.float32)
    return pl.pallas_call(
        _table_repack_body,
        grid=grid,
        in_specs=[in_spec, in_spec],
        out_specs=[out_spec, out_spec],
        out_shape=[out_shape, out_shape],
    )


@functools.lru_cache(maxsize=None)
def _make_sc_gather(NT, F, R):
    info = plsc.get_sparse_core_info()
    NW = info.num_cores * info.num_subcores
    NC = info.num_cores
    rows_per_w = R // NW
    assert rows_per_w * NW == R
    C = 3200
    assert rows_per_w % C == 0
    n_chunks = rows_per_w // C

    mesh = plsc.VectorSubcoreMesh(core_axis_name="c", subcore_axis_name="s")

    @functools.partial(
        pl.kernel,
        mesh=mesh,
        compiler_params=pltpu.CompilerParams(use_tc_tiling_on_sc=False),
        out_type=[
            jax.ShapeDtypeStruct((R, F), jnp.float32),
            jax.ShapeDtypeStruct((R, F), jnp.float32),
        ],
        scratch_types=[
            pltpu.VMEM((C,), jnp.int32),
            pltpu.VMEM((C, F), jnp.float32),
            pltpu.VMEM((C, F), jnp.float32),
            pltpu.SemaphoreType.DMA,
        ],
    )
    def sc_gather(means_hbm, vars_hbm, idx_hbm, means_out, vars_out,
                  idx_v, m_v, v_v, sem):
        wid = lax.axis_index("s") * NC + lax.axis_index("c")
        base = wid * rows_per_w

        def body(ci, carry):
            off = base + ci * C
            pltpu.sync_copy(idx_hbm.at[pl.ds(off, C)], idx_v)
            cm = pltpu.async_copy(means_hbm.at[idx_v], m_v, sem)
            cv = pltpu.async_copy(vars_hbm.at[idx_v], v_v, sem)
            cm.wait()
            cv.wait()
            pltpu.sync_copy(m_v, means_out.at[pl.ds(off, C)])
            pltpu.sync_copy(v_v, vars_out.at[pl.ds(off, C)])
            return carry

        lax.fori_loop(0, n_chunks, body, 0)

    return sc_gather


def _normalize_body(ts_ref, m_ref, v_ref, o_ref, mo_ref, vo_ref):
    LB, F, B = ts_ref.shape
    G = B // 64
    def tr(x):
        z = jnp.transpose(x)
        z5 = z.reshape(64, F, LB, G)
        return jnp.transpose(z5, (2, 1, 0, 3)).reshape(LB, F, B)

    m_t = tr(m_ref[...].reshape(LB * G, 1024))
    v_t = tr(v_ref[...].reshape(LB * G, 1024))
    mo_ref[...] = m_t
    vo_ref[...] = v_t
    o_ref[...] = (ts_ref[...] - m_t) / (jnp.sqrt(v_t) + _EPS)


@functools.lru_cache(maxsize=None)
def _make_normalize(L, F, B, LB):
    grid = (L // LB,)
    G = B // 64
    ts_spec = pl.BlockSpec((LB, F, B), lambda i: (i, 0, 0))
    g_spec = pl.BlockSpec((LB, G, 1024), lambda i: (i, 0, 0))
    shape = jax.ShapeDtypeStruct((L, F, B), jnp.float32)
    return pl.pallas_call(
        _normalize_body,
        grid=grid,
        in_specs=[ts_spec, g_spec, g_spec],
        out_specs=[ts_spec, ts_spec, ts_spec],
        out_shape=[shape, shape, shape],
    )


def kernel(ts_index, window_indices, ts, means_table, vars_table):
    B, L = window_indices.shape
    N, T, F = means_table.shape
    R = B * L
    SB = 128 // F

    mt_nat = jnp.transpose(means_table, (0, 2, 1))
    vt_nat = jnp.transpose(vars_table, (0, 2, 1))
    means_w, vars_w = _make_table_repack(N, T, F, SB)(mt_nat, vt_nat)
    means_flat = means_w.reshape(N * T, F)
    vars_flat = vars_w.reshape(N * T, F)

    s = ts_index.astype(jnp.int32)
    t = jnp.transpose(window_indices).astype(jnp.int32)
    g = ((s // SB) * (T * SB) + (s % SB))[None, :] + t * SB
    g_pi = jnp.transpose(g.reshape(L, 64, B // 64), (0, 2, 1)).reshape(L, B)
    flat_idx = g_pi.reshape(R)

    means_rows, vars_rows = _make_sc_gather(N * T, F, R)(
        means_flat, vars_flat, flat_idx)

    ts_p = jnp.transpose(ts, (1, 2, 0))
    LB = 4
    norm_p, means_p, vars_p = _make_normalize(L, F, B, LB)(
        ts_p,
        means_rows.reshape(L, B // 64, 1024),
        vars_rows.reshape(L, B // 64, 1024))

    return (jnp.transpose(norm_p, (2, 0, 1)),
            jnp.transpose(means_p, (2, 0, 1)),
            jnp.transpose(vars_p, (2, 0, 1)))
